# baseline (speedup 1.0000x reference)
import functools

import jax
import jax.numpy as jnp
from jax import lax
from jax.experimental import pallas as pl
from jax.experimental.pallas import tpu as pltpu
from jax.experimental.pallas import tpu_sc as plsc

_NS = [25000, 20000, 12000, 6000, 3000]
_D = 128
_PAIRS = [(0, 0), (1, 1), (2, 2), (3, 3), (4, 4),
          (0, 1), (0, 2), (0, 3), (0, 4),
          (1, 2), (1, 3), (1, 4),
          (2, 3), (2, 4), (3, 4)]
_NUM_LAYERS = 2
_K = 32
_G = 10
_NB = 4


@functools.lru_cache(maxsize=None)
def _seg_sum(N_src, N_dst, E):
    C0 = (((N_dst + 1) // 2) + 15) // 16 * 16
    N1 = N_dst - C0
    ACC = -(-(C0 + 16) // 256) * 256
    TRASH = ACC - 1
    ZPT = ACC // 16
    ZF, ZR = divmod(ZPT, _K)
    NCH = E // _K
    NBLK = NCH // _G

    def body(x_hbm, src_hbm, dst_hbm, cci_hbm, out_hbm,
             acc_sh, srcs, dsts, ccis, rows, isems, gsem, ssems):
        c = lax.axis_index("c")
        sid = lax.axis_index("s")
        base = c * C0
        half_n = jnp.where(c == 0, C0, N1)

        @pl.loop(0, _K)
        def _zr(e):
            for g in range(_D // 16):
                rows[0][e, pl.ds(g * 16, 16)] = jnp.zeros((16,), jnp.float32)

        zbase = sid * ZPT

        @pl.loop(0, ZF)
        def _za(i):
            pltpu.sync_copy(rows[0], acc_sh.at[pl.ds(zbase + i * _K, _K)])

        if ZR:
            pltpu.sync_copy(rows[0].at[pl.ds(0, ZR)],
                            acc_sh.at[pl.ds(zbase + ZF * _K, ZR)])

        plsc.subcore_barrier()

        @pl.loop(sid, NBLK, step=16)
        def _blk(blk):
            idescs = {}
            gdescs = {}

            def idx_start(j):
                p = j % _NB
                e0 = (blk * _G + j) * _K
                idescs[j] = (
                    pltpu.async_copy(src_hbm.at[pl.ds(e0, _K)], srcs[p],
                                     isems[p]),
                    pltpu.async_copy(dst_hbm.at[pl.ds(e0, _K)], dsts[p],
                                     isems[p]),
                    pltpu.async_copy(cci_hbm.at[pl.ds(e0, _K)], ccis[p],
                                     isems[p]),
                )

            def idx_wait_remap(j):
                p = j % _NB
                for d in idescs.pop(j):
                    d.wait()
                for g in range(_K // 16):
                    sl = pl.ds(g * 16, 16)
                    ld = dsts[p][sl] - base
                    oob = (ld < 0) | (ld >= half_n)
                    dsts[p][sl] = jnp.where(oob, TRASH, ld)

            sdescs = {}

            def gather_start(j):
                p = j % _NB
                gdescs[j] = pltpu.async_copy(x_hbm.at[srcs[p]], rows[p], gsem)

            def scale_scatter(j):
                p = j % _NB
                for gq in range(_K // 16):
                    cv = ccis[p][pl.ds(gq * 16, 16)]
                    for jj in range(16):
                        cs = cv[jj]
                        e = gq * 16 + jj
                        for g in range(_D // 16):
                            sl = pl.ds(g * 16, 16)
                            rows[p][e, sl] = rows[p][e, sl] * cs
                desc = pltpu.make_async_copy(rows[p], acc_sh.at[dsts[p]],
                                             ssems[p])
                desc.start(add=True)
                sdescs[j] = desc

            idx_start(0)
            idx_start(1)
            idx_wait_remap(0)
            gather_start(0)
            for j in range(_G):
                if j >= 2:
                    sdescs.pop(j - 2).wait()
                if j + 2 < _G:
                    idx_start(j + 2)
                if j + 1 < _G:
                    idx_wait_remap(j + 1)
                gdescs.pop(j).wait()
                if j + 1 < _G:
                    gather_start(j + 1)
                scale_scatter(j)
            sdescs.pop(_G - 2).wait()
            sdescs.pop(_G - 1).wait()

        plsc.subcore_barrier()

        @pl.when(sid == 0)
        def _w():
            @pl.when(c == 0)
            def _w0():
                pltpu.sync_copy(acc_sh.at[pl.ds(0, C0)],
                                out_hbm.at[pl.ds(0, C0)])

            @pl.when(c == 1)
            def _w1():
                pltpu.sync_copy(acc_sh.at[pl.ds(0, N1)],
                                out_hbm.at[pl.ds(C0, N1)])

    return pl.kernel(
        body,
        out_type=jax.ShapeDtypeStruct((N_dst, _D), jnp.float32),
        mesh=plsc.VectorSubcoreMesh(core_axis_name="c", subcore_axis_name="s"),
        scratch_types=[
            pltpu.VMEM_SHARED((ACC, _D), jnp.float32),
            [pltpu.VMEM((_K,), jnp.int32) for _ in range(_NB)],
            [pltpu.VMEM((_K,), jnp.int32) for _ in range(_NB)],
            [pltpu.VMEM((_K,), jnp.float32) for _ in range(_NB)],
            [pltpu.VMEM((_K, _D), jnp.float32) for _ in range(_NB)],
            [pltpu.SemaphoreType.DMA for _ in range(_NB)],
            pltpu.SemaphoreType.DMA,
            [pltpu.SemaphoreType.DMA for _ in range(_NB)],
        ],
        name="seg_sum_%d_%d" % (N_dst, E),
    )


@functools.lru_cache(maxsize=None)
def _rank_update(N, k, resid):
    BR = 1000
    grid = (N // BR,)

    def body(*refs):
        aggs = refs[:k]
        ws = refs[k:2 * k]
        acc = jnp.dot(aggs[0][...], ws[0][...],
                      preferred_element_type=jnp.float32)
        for j in range(1, k):
            acc += jnp.dot(aggs[j][...], ws[j][...],
                           preferred_element_type=jnp.float32)
        if resid:
            acc += refs[2 * k][...]
        refs[-1][...] = jnp.maximum(acc, 0.0)

    row_spec = pl.BlockSpec((BR, _D), lambda i: (i, 0))
    w_spec = pl.BlockSpec((_D, _D), lambda i: (0, 0))
    in_specs = [row_spec] * k + [w_spec] * k + ([row_spec] if resid else [])
    return pl.pallas_call(
        body,
        grid=grid,
        in_specs=in_specs,
        out_specs=row_spec,
        out_shape=jax.ShapeDtypeStruct((N, _D), jnp.float32),
    )


def kernel(x_0, x_1, x_2, x_3, x_4, neighborhood_0_to_0, cci_0_to_0, neighborhood_1_to_1, cci_1_to_1, neighborhood_2_to_2, cci_2_to_2, neighborhood_3_to_3, cci_3_to_3, neighborhood_4_to_4, cci_4_to_4, neighborhood_0_to_1, cci_0_to_1, neighborhood_0_to_2, cci_0_to_2, neighborhood_0_to_3, cci_0_to_3, neighborhood_0_to_4, cci_0_to_4, neighborhood_1_to_2, cci_1_to_2, neighborhood_1_to_3, cci_1_to_3, neighborhood_1_to_4, cci_1_to_4, neighborhood_2_to_3, cci_2_to_3, neighborhood_2_to_4, cci_2_to_4, neighborhood_3_to_4, cci_3_to_4, W0_0_0, W0_1_1, W0_2_2, W0_3_3, W0_4_4, W0_0_1, W0_0_2, W0_0_3, W0_0_4, W0_1_2, W0_1_3, W0_1_4, W0_2_3, W0_2_4, W0_3_4, W1_0_0, W1_1_1, W1_2_2, W1_3_3, W1_4_4, W1_0_1, W1_0_2, W1_0_3, W1_0_4, W1_1_2, W1_1_3, W1_1_4, W1_2_3, W1_2_4, W1_3_4):
    inp = dict(locals())
    xs = [inp['x_%d' % i] for i in range(5)]
    nbs = {(s, t): inp['neighborhood_%d_to_%d' % (s, t)] for (s, t) in _PAIRS}
    ccis = {(s, t): inp['cci_%d_to_%d' % (s, t)] for (s, t) in _PAIRS}
    ws = {(l, s, t): inp['W%d_%d_%d' % (l, s, t)]
          for l in range(_NUM_LAYERS) for (s, t) in _PAIRS}

    for l in range(_NUM_LAYERS):
        aggs = {}
        for (s, t) in _PAIRS:
            E = nbs[(s, t)].shape[1]
            aggs[(s, t)] = _seg_sum(_NS[s], _NS[t], E)(
                xs[s], nbs[(s, t)][0], nbs[(s, t)][1], ccis[(s, t)])
        new_xs = []
        for t in range(5):
            srcs = [s for (s, tt) in _PAIRS if tt == t]
            ops = [aggs[(s, t)] for s in srcs] + [ws[(l, s, t)] for s in srcs]
            if l > 0:
                ops.append(xs[t])
            new_xs.append(_rank_update(_NS[t], len(srcs), l > 0)(*ops))
        xs = new_xs
    return tuple(xs)


# speedup vs baseline: 1.2498x; 1.2498x over previous
import functools

import jax
import jax.numpy as jnp
from jax import lax
from jax.experimental import pallas as pl
from jax.experimental.pallas import tpu as pltpu
from jax.experimental.pallas import tpu_sc as plsc

_NS = [25000, 20000, 12000, 6000, 3000]
_D = 128
_PAIRS = [(0, 0), (1, 1), (2, 2), (3, 3), (4, 4),
          (0, 1), (0, 2), (0, 3), (0, 4),
          (1, 2), (1, 3), (1, 4),
          (2, 3), (2, 4), (3, 4)]
_NUM_LAYERS = 2
_K = 64
_G = 10
_NB = 3


@functools.lru_cache(maxsize=None)
def _seg_sum(N_src, N_dst, E):
    C0 = (((N_dst + 1) // 2) + 15) // 16 * 16
    N1 = N_dst - C0
    ACC = -(-(C0 + 16) // 256) * 256
    TRASH = ACC - 1
    ZPT = ACC // 16
    ZF, ZR = divmod(ZPT, _K)
    NCH = E // _K
    NBLK = NCH // _G

    def body(x_hbm, src_hbm, dst_hbm, cci_hbm, out_hbm,
             acc_sh, srcs, dsts, dstc, ccis, rows, isems, gsem, ssems):
        c = lax.axis_index("c")
        sid = lax.axis_index("s")
        base = c * C0
        half_n = jnp.where(c == 0, C0, N1)

        @pl.loop(0, _K)
        def _zr(e):
            for g in range(_D // 16):
                rows[0][e, pl.ds(g * 16, 16)] = jnp.zeros((16,), jnp.float32)

        zbase = sid * ZPT

        @pl.loop(0, ZF)
        def _za(i):
            pltpu.sync_copy(rows[0], acc_sh.at[pl.ds(zbase + i * _K, _K)])

        if ZR:
            pltpu.sync_copy(rows[0].at[pl.ds(0, ZR)],
                            acc_sh.at[pl.ds(zbase + ZF * _K, ZR)])

        plsc.subcore_barrier()

        @pl.loop(sid, NBLK, step=16)
        def _blk(blk):
            idescs = {}
            gdescs = {}

            def idx_start(j):
                p = j % _NB
                e0 = (blk * _G + j) * _K
                idescs[j] = (
                    pltpu.async_copy(src_hbm.at[pl.ds(e0, _K)], srcs[p],
                                     isems[p]),
                    pltpu.async_copy(dst_hbm.at[pl.ds(e0, _K)], dsts[p],
                                     isems[p]),
                    pltpu.async_copy(cci_hbm.at[pl.ds(e0, _K)], ccis[p],
                                     isems[p]),
                )

            def idx_wait_remap(j):
                p = j % _NB
                for d in idescs.pop(j):
                    d.wait()
                for g in range(_K // 16):
                    sl = pl.ds(g * 16, 16)
                    ld = dsts[p][sl] - base
                    oob = (ld < 0) | (ld >= half_n)
                    dstc[p][sl] = jnp.where(oob, TRASH, ld)

            sdescs = {}

            def gather_start(j):
                p = j % _NB
                gdescs[j] = pltpu.async_copy(x_hbm.at[srcs[p]], rows[p], gsem)

            def scale_scatter(j):
                p = j % _NB
                for gq in range(_K // 16):
                    cv = ccis[p][pl.ds(gq * 16, 16)]
                    for jj in range(16):
                        cs = cv[jj]
                        e = gq * 16 + jj
                        for g in range(_D // 16):
                            sl = pl.ds(g * 16, 16)
                            rows[p][e, sl] = rows[p][e, sl] * cs
                desc = pltpu.make_async_copy(rows[p], acc_sh.at[dstc[p]],
                                             ssems[p])
                desc.start(add=True)
                sdescs[j] = desc

            idx_start(0)
            idx_start(1)
            idx_wait_remap(0)
            gather_start(0)
            for j in range(_G):
                if j >= 2:
                    sdescs.pop(j - 2).wait()
                if j + 2 < _G:
                    idx_start(j + 2)
                if j + 1 < _G:
                    idx_wait_remap(j + 1)
                gdescs.pop(j).wait()
                if j + 1 < _G:
                    gather_start(j + 1)
                scale_scatter(j)
            sdescs.pop(_G - 2).wait()
            sdescs.pop(_G - 1).wait()

        plsc.subcore_barrier()

        @pl.when(sid == 0)
        def _w():
            @pl.when(c == 0)
            def _w0():
                pltpu.sync_copy(acc_sh.at[pl.ds(0, C0)],
                                out_hbm.at[pl.ds(0, C0)])

            @pl.when(c == 1)
            def _w1():
                pltpu.sync_copy(acc_sh.at[pl.ds(0, N1)],
                                out_hbm.at[pl.ds(C0, N1)])

    return pl.kernel(
        body,
        out_type=jax.ShapeDtypeStruct((N_dst, _D), jnp.float32),
        mesh=plsc.VectorSubcoreMesh(core_axis_name="c", subcore_axis_name="s"),
        scratch_types=[
            pltpu.VMEM_SHARED((ACC, _D), jnp.float32),
            [pltpu.VMEM((_K,), jnp.int32) for _ in range(_NB)],
            [pltpu.VMEM((_K,), jnp.int32) for _ in range(_NB)],
            [pltpu.VMEM((_K,), jnp.int32) for _ in range(_NB)],
            [pltpu.VMEM((_K,), jnp.float32) for _ in range(_NB)],
            [pltpu.VMEM((_K, _D), jnp.float32) for _ in range(_NB)],
            [pltpu.SemaphoreType.DMA for _ in range(_NB)],
            pltpu.SemaphoreType.DMA,
            [pltpu.SemaphoreType.DMA for _ in range(_NB)],
        ],
        name="seg_sum_%d_%d" % (N_dst, E),
    )


@functools.lru_cache(maxsize=None)
def _rank_update(N, k, resid):
    BR = 1000
    grid = (N // BR,)

    def body(*refs):
        aggs = refs[:k]
        ws = refs[k:2 * k]
        acc = jnp.dot(aggs[0][...], ws[0][...],
                      preferred_element_type=jnp.float32)
        for j in range(1, k):
            acc += jnp.dot(aggs[j][...], ws[j][...],
                           preferred_element_type=jnp.float32)
        if resid:
            acc += refs[2 * k][...]
        refs[-1][...] = jnp.maximum(acc, 0.0)

    row_spec = pl.BlockSpec((BR, _D), lambda i: (i, 0))
    w_spec = pl.BlockSpec((_D, _D), lambda i: (0, 0))
    in_specs = [row_spec] * k + [w_spec] * k + ([row_spec] if resid else [])
    return pl.pallas_call(
        body,
        grid=grid,
        in_specs=in_specs,
        out_specs=row_spec,
        out_shape=jax.ShapeDtypeStruct((N, _D), jnp.float32),
    )


def kernel(x_0, x_1, x_2, x_3, x_4, neighborhood_0_to_0, cci_0_to_0, neighborhood_1_to_1, cci_1_to_1, neighborhood_2_to_2, cci_2_to_2, neighborhood_3_to_3, cci_3_to_3, neighborhood_4_to_4, cci_4_to_4, neighborhood_0_to_1, cci_0_to_1, neighborhood_0_to_2, cci_0_to_2, neighborhood_0_to_3, cci_0_to_3, neighborhood_0_to_4, cci_0_to_4, neighborhood_1_to_2, cci_1_to_2, neighborhood_1_to_3, cci_1_to_3, neighborhood_1_to_4, cci_1_to_4, neighborhood_2_to_3, cci_2_to_3, neighborhood_2_to_4, cci_2_to_4, neighborhood_3_to_4, cci_3_to_4, W0_0_0, W0_1_1, W0_2_2, W0_3_3, W0_4_4, W0_0_1, W0_0_2, W0_0_3, W0_0_4, W0_1_2, W0_1_3, W0_1_4, W0_2_3, W0_2_4, W0_3_4, W1_0_0, W1_1_1, W1_2_2, W1_3_3, W1_4_4, W1_0_1, W1_0_2, W1_0_3, W1_0_4, W1_1_2, W1_1_3, W1_1_4, W1_2_3, W1_2_4, W1_3_4):
    inp = dict(locals())
    xs = [inp['x_%d' % i] for i in range(5)]
    nbs = {(s, t): inp['neighborhood_%d_to_%d' % (s, t)] for (s, t) in _PAIRS}
    ccis = {(s, t): inp['cci_%d_to_%d' % (s, t)] for (s, t) in _PAIRS}
    ws = {(l, s, t): inp['W%d_%d_%d' % (l, s, t)]
          for l in range(_NUM_LAYERS) for (s, t) in _PAIRS}

    for l in range(_NUM_LAYERS):
        aggs = {}
        for (s, t) in _PAIRS:
            E = nbs[(s, t)].shape[1]
            aggs[(s, t)] = _seg_sum(_NS[s], _NS[t], E)(
                xs[s], nbs[(s, t)][0], nbs[(s, t)][1], ccis[(s, t)])
        new_xs = []
        for t in range(5):
            srcs = [s for (s, tt) in _PAIRS if tt == t]
            ops = [aggs[(s, t)] for s in srcs] + [ws[(l, s, t)] for s in srcs]
            if l > 0:
                ops.append(xs[t])
            new_xs.append(_rank_update(_NS[t], len(srcs), l > 0)(*ops))
        xs = new_xs
    return tuple(xs)
